# baseline (speedup 1.0000x reference)
import functools

import jax
import jax.numpy as jnp
import numpy as np
from jax import lax
from jax.experimental import pallas as pl
from jax.experimental.pallas import tpu as pltpu
from jax.experimental.pallas import tpu_sc as plsc

USER_NUM = 2000
ITEM_NUM = 8000
LATDIM = 256
GCN_LAYER = 3
N_EDGES = 160000
N_NODES = USER_NUM + ITEM_NUM

NSUB = 16
NW = LATDIM // 32
K = 32
KI = LATDIM // 2
NHALF = 5120
NPAD = 2 * NHALF
FROWS = 2 * NPAD
RPT = 2 * NHALF // NSUB
RBLK = 16
NRB = RPT // RBLK
EGRAN = 2 * NSUB * K
CAP = ((N_EDGES + EGRAN - 1) // EGRAN) * EGRAN

_PERM = np.arange(LATDIM).reshape(NW, 2, 16).transpose(0, 2, 1).reshape(-1)


def _layer_body(last, cnt_hbm, cpk_hbm, vpk_hbm, rpk_hbm, tsrc_hbm, fsrc_hbm,
                f_hbm, tnext_hbm,
                acc, cntbuf, cbuf, vbuf, rbuf, rbA, rbB,
                rsb0, rsb1, rsfA0, rsfA1, rsfB0, rsfB1,
                abuf, fbuf, tbuf, zbuf,
                semi, semg, semsc):
    c = lax.axis_index("c")
    s = lax.axis_index("s")
    r0base = s * RPT
    f0base = c * 2 * NHALF + s * RPT
    t0base = c * NHALF + s * (RPT // 2)

    zv = jnp.zeros((16,), jnp.float32)

    pltpu.sync_copy(cnt_hbm, cntbuf)
    cv = cntbuf[pl.ds(0, 16)]
    nc = jnp.where(c == 0, cv[0], cv[1])
    nj = lax.shift_right_logical(nc, 9)
    ebase = c * CAP + s * nj * K

    def start_idx(t, slot):
        eb = pl.multiple_of(ebase + t * K, 8)
        pltpu.async_copy(cpk_hbm.at[pl.ds(eb, K)], cbuf.at[slot], semi)
        pltpu.async_copy(vpk_hbm.at[pl.ds(eb, K)], vbuf.at[slot], semi)
        pltpu.async_copy(rpk_hbm.at[pl.ds(eb, K)], rbuf.at[slot], semi)

    def wait_idx(slot):
        pltpu.make_async_copy(
            cpk_hbm.at[pl.ds(0, K)], cbuf.at[slot], semi).wait()
        pltpu.make_async_copy(
            vpk_hbm.at[pl.ds(0, K)], vbuf.at[slot], semi).wait()
        pltpu.make_async_copy(
            rpk_hbm.at[pl.ds(0, K)], rbuf.at[slot], semi).wait()
        for g in range(K // 16):
            rv = rbuf[slot, pl.ds(g * 16, 16)]
            rbA[slot, pl.ds(g * 16, 16)] = rv * 2
            rbB[slot, pl.ds(g * 16, 16)] = rv * 2 + 1

    def start_gather(slot):
        rsb = rsb0 if slot == 0 else rsb1
        pltpu.async_copy(tsrc_hbm.at[cbuf.at[slot]], rsb, semg)

    def wait_gather(slot):
        rsb = rsb0 if slot == 0 else rsb1
        pltpu.make_async_copy(tsrc_hbm.at[cbuf.at[slot]], rsb, semg).wait()

    def start_scatter(slot):
        rsfA = rsfA0 if slot == 0 else rsfA1
        rsfB = rsfB0 if slot == 0 else rsfB1
        pltpu.async_copy(rsfA, acc.at[rbA.at[slot]], semsc, add=True)
        pltpu.async_copy(rsfB, acc.at[rbB.at[slot]], semsc, add=True)

    def wait_scatter(slot):
        rsfA = rsfA0 if slot == 0 else rsfA1
        rsfB = rsfB0 if slot == 0 else rsfB1
        pltpu.make_async_copy(rsfA, acc.at[rbA.at[slot]], semsc).wait()
        pltpu.make_async_copy(rsfB, acc.at[rbB.at[slot]], semsc).wait()

    def scale_chunk(slot):
        rsb = rsb0 if slot == 0 else rsb1
        rsfA = rsfA0 if slot == 0 else rsfA1
        rsfB = rsfB0 if slot == 0 else rsfB1

        def scale(g, ecarry):
            vv = vbuf[slot, pl.ds(g * 16, 16)]
            for e16 in range(16):
                e = g * 16 + e16
                sv = vv[e16]
                for gg in range(NW):
                    va = plsc.bitcast(
                        rsb[e, pl.ds(gg * 16, 16)], jnp.bfloat16)
                    a, b = plsc.unpack(
                        va, format=plsc.PackFormat.INTERLEAVED)
                    rsf = rsfA if gg < NW // 2 else rsfB
                    g2 = gg if gg < NW // 2 else gg - NW // 2
                    rsf[e, pl.ds(g2 * 32, 16)] = a * sv
                    rsf[e, pl.ds(g2 * 32 + 16, 16)] = b * sv
            return ecarry

        lax.fori_loop(0, K // 16, scale, 0)

    def zinit(i, carry):
        for j in range(128 // 16):
            zbuf[i, pl.ds(j * 16, 16)] = zv
        return carry

    lax.fori_loop(0, RBLK, zinit, 0)

    def zero_blk(b, carry):
        pltpu.sync_copy(zbuf, acc.at[pl.ds(r0base + b * RBLK, RBLK)])
        return carry

    lax.fori_loop(0, NRB, zero_blk, 0)
    plsc.subcore_barrier()

    @pl.when(nj > 0)
    def _():
        start_idx(0, 0)
        wait_idx(0)
        start_gather(0)

    def pipe(i, carry):
        for b in range(2):
            t = 2 * i + b
            wait_gather(b)

            @pl.when(t >= 1)
            def _():
                wait_scatter(1 - b)

            @pl.when(t + 1 < nj)
            def _():
                start_idx(t + 1, 1 - b)
                wait_idx(1 - b)
                start_gather(1 - b)

            scale_chunk(b)
            start_scatter(b)
        return carry

    lax.fori_loop(0, lax.shift_right_logical(nj, 1), pipe, 0)

    @pl.when(nj > 0)
    def _():
        wait_scatter(1)

    plsc.subcore_barrier()

    def readback(b, carry):
        r0 = r0base + b * RBLK
        f0 = f0base + b * RBLK
        t0 = t0base + b * (RBLK // 2)
        pltpu.sync_copy(acc.at[pl.ds(r0, RBLK)], abuf)
        pltpu.sync_copy(fsrc_hbm.at[pl.ds(f0, RBLK)], fbuf)

        def addrow(i, icarry):
            for j in range(128 // 16):
                fbuf[i, pl.ds(j * 16, 16)] = (
                    fbuf[i, pl.ds(j * 16, 16)]
                    + abuf[i, pl.ds(j * 16, 16)])
            return icarry

        lax.fori_loop(0, RBLK, addrow, 0)

        if not last:
            def packrow(i, icarry):
                for gg in range(NW):
                    src_row = 2 * i + (0 if gg < NW // 2 else 1)
                    g2 = gg if gg < NW // 2 else gg - NW // 2
                    a = abuf[src_row, pl.ds(g2 * 32, 16)]
                    bb = abuf[src_row, pl.ds(g2 * 32 + 16, 16)]
                    tbuf[i, pl.ds(gg * 16, 16)] = plsc.bitcast(
                        plsc.pack(
                            a, bb, format=plsc.PackFormat.INTERLEAVED),
                        jnp.int32)
                return icarry

            lax.fori_loop(0, RBLK // 2, packrow, 0)

        pltpu.sync_copy(fbuf, f_hbm.at[pl.ds(f0, RBLK)])
        if not last:
            pltpu.sync_copy(tbuf, tnext_hbm.at[pl.ds(t0, RBLK // 2)])
        return carry

    lax.fori_loop(0, NRB, readback, 0)


def _make_layer(last):
    mesh = plsc.VectorSubcoreMesh(core_axis_name="c", subcore_axis_name="s")
    return functools.partial(
        pl.kernel,
        mesh=mesh,
        compiler_params=pltpu.CompilerParams(needs_layout_passes=False),
        out_type=(
            jax.ShapeDtypeStruct((FROWS, 128), jnp.float32),
            jax.ShapeDtypeStruct((NPAD, KI), jnp.int32),
        ),
        scratch_types=[
            pltpu.VMEM_SHARED((2 * NHALF, 128), jnp.float32),
            pltpu.VMEM((16,), jnp.int32),
            pltpu.VMEM((2, K), jnp.int32),
            pltpu.VMEM((2, K), jnp.float32),
            pltpu.VMEM((2, K), jnp.int32),
            pltpu.VMEM((2, K), jnp.int32),
            pltpu.VMEM((2, K), jnp.int32),
            pltpu.VMEM((K, KI), jnp.int32),
            pltpu.VMEM((K, KI), jnp.int32),
            pltpu.VMEM((K, 128), jnp.float32),
            pltpu.VMEM((K, 128), jnp.float32),
            pltpu.VMEM((K, 128), jnp.float32),
            pltpu.VMEM((K, 128), jnp.float32),
            pltpu.VMEM((RBLK, 128), jnp.float32),
            pltpu.VMEM((RBLK, 128), jnp.float32),
            pltpu.VMEM((RBLK // 2, KI), jnp.int32),
            pltpu.VMEM((RBLK, 128), jnp.float32),
            pltpu.SemaphoreType.DMA,
            pltpu.SemaphoreType.DMA,
            pltpu.SemaphoreType.DMA,
        ],
    )(functools.partial(_layer_body, last))


@jax.jit
def _run(cnt, cpk, vpk, rpk, tin, finit):
    mid = _make_layer(False)
    end = _make_layer(True)
    f1, t1 = mid(cnt, cpk, vpk, rpk, tin, finit)
    f2, t2 = mid(cnt, cpk, vpk, rpk, t1, f1)
    f3, _ = end(cnt, cpk, vpk, rpk, t2, f2)
    return f3


def kernel(edge_index, edge_vals, user_embeds, item_embeds):
    embeds = jnp.concatenate([user_embeds, item_embeds], axis=0)
    tstd = jnp.zeros((NPAD, LATDIM), jnp.float32).at[:N_NODES].set(embeds)
    finit = tstd.reshape(FROWS, 128)
    tin_bf = jnp.take(tstd, jnp.asarray(_PERM), axis=1).astype(jnp.bfloat16)
    tin = jax.lax.bitcast_convert_type(
        tin_bf.reshape(NPAD, KI, 2), jnp.int32)
    row = edge_index[0]
    col = edge_index[1]
    m0 = row < NHALF
    i0 = jnp.cumsum(m0.astype(jnp.int32))
    i1 = jnp.cumsum(1 - m0.astype(jnp.int32))
    n0 = i0[-1]
    pos = jnp.where(m0, i0 - 1, CAP + i1 - 1)
    cpk = jnp.zeros((2 * CAP,), jnp.int32).at[pos].set(col)
    rpk = jnp.zeros((2 * CAP,), jnp.int32).at[pos].set(
        row - jnp.where(m0, 0, NHALF))
    vpk = jnp.zeros((2 * CAP,), jnp.float32).at[pos].set(edge_vals)
    n0p = ((n0 + EGRAN - 1) // EGRAN) * EGRAN
    n1p = ((N_EDGES - n0 + EGRAN - 1) // EGRAN) * EGRAN
    cnt = jnp.zeros((16,), jnp.int32).at[0].set(n0p).at[1].set(n1p)
    f = _run(cnt, cpk, vpk, rpk, tin, finit)
    final = f[:2 * N_NODES].reshape(N_NODES, LATDIM)
    return final[:USER_NUM], final[USER_NUM:]


# speedup vs baseline: 1.6736x; 1.6736x over previous
import functools

import jax
import jax.numpy as jnp
from jax import lax
from jax.experimental import pallas as pl
from jax.experimental.pallas import tpu as pltpu
from jax.experimental.pallas import tpu_sc as plsc

USER_NUM = 2000
ITEM_NUM = 8000
LATDIM = 256
GCN_LAYER = 3
N_EDGES = 160000
N_NODES = USER_NUM + ITEM_NUM

NSUB = 16
NW = LATDIM // 32
K = 32
KI = LATDIM // 2
NHALF = 5120
NPAD = 2 * NHALF
FROWS = 2 * NPAD
RPT = 2 * NHALF // NSUB
RBLK = 16
NRB = RPT // RBLK
EPT = 10240
EPAD = NSUB * EPT
REG = EPT + 2 * K
NREG = 2 * NSUB * REG
PB = 2048
NPB = EPT // PB


def _part_body(rowp_hbm, colp_hbm, valp_hbm,
               cpk_hbm, rpk_hbm, vpk_hbm, cnt_hbm,
               rin, cin, vin, ccomp, rcomp, vcomp, cw):
    c = lax.axis_index("c")
    s = lax.axis_index("s")
    wid = c * NSUB + s
    lo = c * NHALF
    zi = jnp.zeros((16,), jnp.int32)
    zf = jnp.zeros((16,), jnp.float32)

    def block(bi, off):
        eb = pl.multiple_of(s * EPT + bi * PB, 8)
        pltpu.sync_copy(rowp_hbm.at[pl.ds(eb, PB)], rin)
        pltpu.sync_copy(colp_hbm.at[pl.ds(eb, PB)], cin)
        pltpu.sync_copy(valp_hbm.at[pl.ds(eb, PB)], vin)

        def group(g, off2):
            rv = rin[pl.ds(g * 16, 16)] - lo
            m = (rv >= 0) & (rv < NHALF)
            plsc.store_compressed(rcomp.at[pl.ds(off2, 16)], rv, mask=m)
            plsc.store_compressed(
                ccomp.at[pl.ds(off2, 16)], cin[pl.ds(g * 16, 16)], mask=m)
            plsc.store_compressed(
                vcomp.at[pl.ds(off2, 16)], vin[pl.ds(g * 16, 16)], mask=m)
            npop = plsc.all_reduce_population_count(m)
            return off2 + npop[0]

        return lax.fori_loop(0, PB // 16, group, off)

    off = lax.fori_loop(0, NPB, block, jnp.int32(0))

    for gp in range(2 * K // 16):
        ccomp[pl.ds(off + gp * 16, 16)] = zi
        rcomp[pl.ds(off + gp * 16, 16)] = zi
        vcomp[pl.ds(off + gp * 16, 16)] = zf
    ncp = lax.shift_left(
        lax.shift_right_logical(off + 2 * K - 1, 6), 6)
    cw[pl.ds(0, 16)] = jnp.full((16,), 1, jnp.int32) * ncp

    rb = wid * REG
    pltpu.sync_copy(ccomp, cpk_hbm.at[pl.ds(rb, REG)])
    pltpu.sync_copy(rcomp, rpk_hbm.at[pl.ds(rb, REG)])
    pltpu.sync_copy(vcomp, vpk_hbm.at[pl.ds(rb, REG)])
    pltpu.sync_copy(cw, cnt_hbm.at[pl.ds(wid * 16, 16)])


def _layer_body(last, cnt_hbm, cpk_hbm, vpk_hbm, rpk_hbm, tsrc_hbm, fsrc_hbm,
                f_hbm, tnext_hbm,
                acc, cntbuf, cbuf, vbuf, rbuf, rbA, rbB,
                rsb0, rsb1, rsfA0, rsfA1, rsfB0, rsfB1,
                abuf, fbuf, tbuf, zbuf,
                semi, semg, semsc):
    c = lax.axis_index("c")
    s = lax.axis_index("s")
    wid = c * NSUB + s
    r0base = s * RPT
    f0base = c * 2 * NHALF + s * RPT
    t0base = c * NHALF + s * (RPT // 2)

    zv = jnp.zeros((16,), jnp.float32)

    pltpu.sync_copy(cnt_hbm.at[pl.ds(wid * 16, 16)], cntbuf)
    cv = cntbuf[pl.ds(0, 16)]
    nc = cv[0]
    nj = lax.shift_right_logical(nc, 5)
    ebase = wid * REG

    def start_idx(t, slot):
        eb = pl.multiple_of(ebase + t * K, 8)
        pltpu.async_copy(cpk_hbm.at[pl.ds(eb, K)], cbuf.at[slot], semi)
        pltpu.async_copy(vpk_hbm.at[pl.ds(eb, K)], vbuf.at[slot], semi)
        pltpu.async_copy(rpk_hbm.at[pl.ds(eb, K)], rbuf.at[slot], semi)

    def wait_idx(slot):
        pltpu.make_async_copy(
            cpk_hbm.at[pl.ds(0, K)], cbuf.at[slot], semi).wait()
        pltpu.make_async_copy(
            vpk_hbm.at[pl.ds(0, K)], vbuf.at[slot], semi).wait()
        pltpu.make_async_copy(
            rpk_hbm.at[pl.ds(0, K)], rbuf.at[slot], semi).wait()
        for g in range(K // 16):
            rv = rbuf[slot, pl.ds(g * 16, 16)]
            rbA[slot, pl.ds(g * 16, 16)] = rv * 2
            rbB[slot, pl.ds(g * 16, 16)] = rv * 2 + 1

    def start_gather(slot):
        rsb = rsb0 if slot == 0 else rsb1
        pltpu.async_copy(tsrc_hbm.at[cbuf.at[slot]], rsb, semg)

    def wait_gather(slot):
        rsb = rsb0 if slot == 0 else rsb1
        pltpu.make_async_copy(tsrc_hbm.at[cbuf.at[slot]], rsb, semg).wait()

    def start_scatter(slot):
        rsfA = rsfA0 if slot == 0 else rsfA1
        rsfB = rsfB0 if slot == 0 else rsfB1
        pltpu.async_copy(rsfA, acc.at[rbA.at[slot]], semsc, add=True)
        pltpu.async_copy(rsfB, acc.at[rbB.at[slot]], semsc, add=True)

    def wait_scatter(slot):
        rsfA = rsfA0 if slot == 0 else rsfA1
        rsfB = rsfB0 if slot == 0 else rsfB1
        pltpu.make_async_copy(rsfA, acc.at[rbA.at[slot]], semsc).wait()
        pltpu.make_async_copy(rsfB, acc.at[rbB.at[slot]], semsc).wait()

    def scale_chunk(slot):
        rsb = rsb0 if slot == 0 else rsb1
        rsfA = rsfA0 if slot == 0 else rsfA1
        rsfB = rsfB0 if slot == 0 else rsfB1

        def scale(g, ecarry):
            vv = vbuf[slot, pl.ds(g * 16, 16)]
            for e16 in range(16):
                e = g * 16 + e16
                sv = vv[e16]
                for gg in range(NW):
                    va = plsc.bitcast(
                        rsb[e, pl.ds(gg * 16, 16)], jnp.bfloat16)
                    a, b = plsc.unpack(
                        va, format=plsc.PackFormat.INTERLEAVED)
                    rsf = rsfA if gg < NW // 2 else rsfB
                    g2 = gg if gg < NW // 2 else gg - NW // 2
                    rsf[e, pl.ds(g2 * 32, 16)] = a * sv
                    rsf[e, pl.ds(g2 * 32 + 16, 16)] = b * sv
            return ecarry

        lax.fori_loop(0, K // 16, scale, 0)

    def zinit(i, carry):
        for j in range(128 // 16):
            zbuf[i, pl.ds(j * 16, 16)] = zv
        return carry

    lax.fori_loop(0, RBLK, zinit, 0)

    def zero_blk(b, carry):
        pltpu.sync_copy(zbuf, acc.at[pl.ds(r0base + b * RBLK, RBLK)])
        return carry

    lax.fori_loop(0, NRB, zero_blk, 0)
    plsc.subcore_barrier()

    @pl.when(nj > 0)
    def _():
        start_idx(0, 0)
        wait_idx(0)
        start_gather(0)

    def pipe(i, carry):
        for b in range(2):
            t = 2 * i + b
            wait_gather(b)

            @pl.when(t >= 1)
            def _():
                wait_scatter(1 - b)

            @pl.when(t + 1 < nj)
            def _():
                start_idx(t + 1, 1 - b)
                wait_idx(1 - b)
                start_gather(1 - b)

            scale_chunk(b)
            start_scatter(b)
        return carry

    lax.fori_loop(0, lax.shift_right_logical(nj, 1), pipe, 0)

    @pl.when(nj > 0)
    def _():
        wait_scatter(1)

    plsc.subcore_barrier()

    def readback(b, carry):
        r0 = r0base + b * RBLK
        f0 = f0base + b * RBLK
        t0 = t0base + b * (RBLK // 2)
        pltpu.sync_copy(acc.at[pl.ds(r0, RBLK)], abuf)
        pltpu.sync_copy(fsrc_hbm.at[pl.ds(f0, RBLK)], fbuf)

        def addrow(i, icarry):
            for j in range(128 // 16):
                fbuf[i, pl.ds(j * 16, 16)] = (
                    fbuf[i, pl.ds(j * 16, 16)]
                    + abuf[i, pl.ds(j * 16, 16)])
            return icarry

        lax.fori_loop(0, RBLK, addrow, 0)

        if not last:
            def packrow(i, icarry):
                for gg in range(NW):
                    src_row = 2 * i + (0 if gg < NW // 2 else 1)
                    g2 = gg if gg < NW // 2 else gg - NW // 2
                    a = abuf[src_row, pl.ds(g2 * 32, 16)]
                    bb = abuf[src_row, pl.ds(g2 * 32 + 16, 16)]
                    tbuf[i, pl.ds(gg * 16, 16)] = plsc.bitcast(
                        plsc.pack(
                            a, bb, format=plsc.PackFormat.INTERLEAVED),
                        jnp.int32)
                return icarry

            lax.fori_loop(0, RBLK // 2, packrow, 0)

        pltpu.sync_copy(fbuf, f_hbm.at[pl.ds(f0, RBLK)])
        if not last:
            pltpu.sync_copy(tbuf, tnext_hbm.at[pl.ds(t0, RBLK // 2)])
        return carry

    lax.fori_loop(0, NRB, readback, 0)


_MESH = plsc.VectorSubcoreMesh(core_axis_name="c", subcore_axis_name="s")
_CP = pltpu.CompilerParams(needs_layout_passes=False)


def _make_part():
    return functools.partial(
        pl.kernel,
        mesh=_MESH,
        compiler_params=_CP,
        out_type=(
            jax.ShapeDtypeStruct((NREG,), jnp.int32),
            jax.ShapeDtypeStruct((NREG,), jnp.int32),
            jax.ShapeDtypeStruct((NREG,), jnp.float32),
            jax.ShapeDtypeStruct((2 * NSUB * 16,), jnp.int32),
        ),
        scratch_types=[
            pltpu.VMEM((PB,), jnp.int32),
            pltpu.VMEM((PB,), jnp.int32),
            pltpu.VMEM((PB,), jnp.float32),
            pltpu.VMEM((REG,), jnp.int32),
            pltpu.VMEM((REG,), jnp.int32),
            pltpu.VMEM((REG,), jnp.float32),
            pltpu.VMEM((16,), jnp.int32),
        ],
    )(_part_body)


def _make_layer(last):
    return functools.partial(
        pl.kernel,
        mesh=_MESH,
        compiler_params=_CP,
        out_type=(
            jax.ShapeDtypeStruct((FROWS, 128), jnp.float32),
            jax.ShapeDtypeStruct((NPAD, KI), jnp.int32),
        ),
        scratch_types=[
            pltpu.VMEM_SHARED((2 * NHALF, 128), jnp.float32),
            pltpu.VMEM((16,), jnp.int32),
            pltpu.VMEM((2, K), jnp.int32),
            pltpu.VMEM((2, K), jnp.float32),
            pltpu.VMEM((2, K), jnp.int32),
            pltpu.VMEM((2, K), jnp.int32),
            pltpu.VMEM((2, K), jnp.int32),
            pltpu.VMEM((K, KI), jnp.int32),
            pltpu.VMEM((K, KI), jnp.int32),
            pltpu.VMEM((K, 128), jnp.float32),
            pltpu.VMEM((K, 128), jnp.float32),
            pltpu.VMEM((K, 128), jnp.float32),
            pltpu.VMEM((K, 128), jnp.float32),
            pltpu.VMEM((RBLK, 128), jnp.float32),
            pltpu.VMEM((RBLK, 128), jnp.float32),
            pltpu.VMEM((RBLK // 2, KI), jnp.int32),
            pltpu.VMEM((RBLK, 128), jnp.float32),
            pltpu.SemaphoreType.DMA,
            pltpu.SemaphoreType.DMA,
            pltpu.SemaphoreType.DMA,
        ],
    )(functools.partial(_layer_body, last))


@jax.jit
def _run(rowp, colp, valp, tin, finit):
    part = _make_part()
    mid = _make_layer(False)
    end = _make_layer(True)
    cpk, rpk, vpk, cnt = part(rowp, colp, valp)
    f1, t1 = mid(cnt, cpk, vpk, rpk, tin, finit)
    f2, t2 = mid(cnt, cpk, vpk, rpk, t1, f1)
    f3, _ = end(cnt, cpk, vpk, rpk, t2, f2)
    return f3


def kernel(edge_index, edge_vals, user_embeds, item_embeds):
    embeds = jnp.concatenate([user_embeds, item_embeds], axis=0)
    tstd = jnp.zeros((NPAD, LATDIM), jnp.float32).at[:N_NODES].set(embeds)
    finit = tstd.reshape(FROWS, 128)
    tin_bf = (tstd.reshape(NPAD, NW, 2, 16).transpose(0, 1, 3, 2)
              .reshape(NPAD, LATDIM).astype(jnp.bfloat16))
    tin = jax.lax.bitcast_convert_type(
        tin_bf.reshape(NPAD, KI, 2), jnp.int32)
    pad = EPAD - N_EDGES
    rowp = jnp.pad(edge_index[0], (0, pad))
    colp = jnp.pad(edge_index[1], (0, pad))
    valp = jnp.pad(edge_vals, (0, pad))
    f = _run(rowp, colp, valp, tin, finit)
    final = f[:2 * N_NODES].reshape(N_NODES, LATDIM)
    return final[:USER_NUM], final[USER_NUM:]


# speedup vs baseline: 2.5564x; 1.5275x over previous
import functools

import jax
import jax.numpy as jnp
from jax import lax
from jax.experimental import pallas as pl
from jax.experimental.pallas import tpu as pltpu
from jax.experimental.pallas import tpu_sc as plsc

USER_NUM = 2000
ITEM_NUM = 8000
LATDIM = 256
GCN_LAYER = 3
N_EDGES = 160000
N_NODES = USER_NUM + ITEM_NUM

NSUB = 16
NW = LATDIM // 32
K = 32
KI = LATDIM // 2
NHALF = 5120
NPAD = 2 * NHALF
FROWS = 2 * NPAD
RPT = 2 * NHALF // NSUB
RBLK = 16
NRB = RPT // RBLK
EPT = 10240
EPAD = NSUB * EPT
REG = EPT + 2 * K
NREG = 2 * NSUB * REG
PB = 2048
NPB = EPT // PB


def _part_body(rowp_hbm, colp_hbm, valp_hbm,
               cpk_hbm, rpk_hbm, vpk_hbm, cnt_hbm,
               rin, cin, vin, ccomp, rcomp, vcomp, cw):
    c = lax.axis_index("c")
    s = lax.axis_index("s")
    wid = c * NSUB + s
    lo = c * NHALF
    zi = jnp.zeros((16,), jnp.int32)
    zf = jnp.zeros((16,), jnp.float32)

    def block(bi, off):
        eb = pl.multiple_of(s * EPT + bi * PB, 8)
        pltpu.sync_copy(rowp_hbm.at[pl.ds(eb, PB)], rin)
        pltpu.sync_copy(colp_hbm.at[pl.ds(eb, PB)], cin)
        pltpu.sync_copy(valp_hbm.at[pl.ds(eb, PB)], vin)

        def group(g, off2):
            rv = rin[pl.ds(g * 16, 16)] - lo
            m = (rv >= 0) & (rv < NHALF)
            plsc.store_compressed(rcomp.at[pl.ds(off2, 16)], rv, mask=m)
            plsc.store_compressed(
                ccomp.at[pl.ds(off2, 16)], cin[pl.ds(g * 16, 16)], mask=m)
            plsc.store_compressed(
                vcomp.at[pl.ds(off2, 16)], vin[pl.ds(g * 16, 16)], mask=m)
            npop = plsc.all_reduce_population_count(m)
            return off2 + npop[0]

        return lax.fori_loop(0, PB // 16, group, off)

    off = lax.fori_loop(0, NPB, block, jnp.int32(0))

    lanes = lax.iota(jnp.int32, 16)
    for gp in range(2 * K // 16):
        spread = lanes + (s * (2 * K) + gp * 16)
        ccomp[pl.ds(off + gp * 16, 16)] = spread
        rcomp[pl.ds(off + gp * 16, 16)] = spread
        vcomp[pl.ds(off + gp * 16, 16)] = zf
    ncp = lax.shift_left(
        lax.shift_right_logical(off + 2 * K - 1, 6), 6)
    cw[pl.ds(0, 16)] = jnp.full((16,), 1, jnp.int32) * ncp

    rb = wid * REG
    pltpu.sync_copy(ccomp, cpk_hbm.at[pl.ds(rb, REG)])
    pltpu.sync_copy(rcomp, rpk_hbm.at[pl.ds(rb, REG)])
    pltpu.sync_copy(vcomp, vpk_hbm.at[pl.ds(rb, REG)])
    pltpu.sync_copy(cw, cnt_hbm.at[pl.ds(wid * 16, 16)])


def _layer_body(last, cnt_hbm, cpk_hbm, vpk_hbm, rpk_hbm, tsrc_hbm, fsrc_hbm,
                f_hbm, tnext_hbm,
                acc, cntbuf, cbuf, vbuf, rbuf, rbA, rbB,
                rsb0, rsb1, rsfA0, rsfA1, rsfB0, rsfB1,
                abuf, fbuf, tbuf, zbuf,
                semi, semg, semsc):
    c = lax.axis_index("c")
    s = lax.axis_index("s")
    wid = c * NSUB + s
    r0base = s * RPT
    f0base = c * 2 * NHALF + s * RPT
    t0base = c * NHALF + s * (RPT // 2)

    zv = jnp.zeros((16,), jnp.float32)

    pltpu.sync_copy(cnt_hbm.at[pl.ds(wid * 16, 16)], cntbuf)
    cv = cntbuf[pl.ds(0, 16)]
    nc = cv[0]
    nj = lax.shift_right_logical(nc, 5)
    ebase = wid * REG

    def start_idx(t, slot):
        eb = pl.multiple_of(ebase + t * K, 8)
        pltpu.async_copy(cpk_hbm.at[pl.ds(eb, K)], cbuf.at[slot], semi)
        pltpu.async_copy(vpk_hbm.at[pl.ds(eb, K)], vbuf.at[slot], semi)
        pltpu.async_copy(rpk_hbm.at[pl.ds(eb, K)], rbuf.at[slot], semi)

    def wait_idx(slot):
        pltpu.make_async_copy(
            cpk_hbm.at[pl.ds(0, K)], cbuf.at[slot], semi).wait()
        pltpu.make_async_copy(
            vpk_hbm.at[pl.ds(0, K)], vbuf.at[slot], semi).wait()
        pltpu.make_async_copy(
            rpk_hbm.at[pl.ds(0, K)], rbuf.at[slot], semi).wait()
        for g in range(K // 16):
            rv = rbuf[slot, pl.ds(g * 16, 16)]
            rbA[slot, pl.ds(g * 16, 16)] = rv * 2
            rbB[slot, pl.ds(g * 16, 16)] = rv * 2 + 1

    def start_gather(slot):
        rsb = rsb0 if slot == 0 else rsb1
        pltpu.async_copy(tsrc_hbm.at[cbuf.at[slot]], rsb, semg)

    def wait_gather(slot):
        rsb = rsb0 if slot == 0 else rsb1
        pltpu.make_async_copy(tsrc_hbm.at[cbuf.at[slot]], rsb, semg).wait()

    def start_scatter(slot):
        rsfA = rsfA0 if slot == 0 else rsfA1
        rsfB = rsfB0 if slot == 0 else rsfB1
        pltpu.async_copy(rsfA, acc.at[rbA.at[slot]], semsc, add=True)
        pltpu.async_copy(rsfB, acc.at[rbB.at[slot]], semsc, add=True)

    def wait_scatter(slot):
        rsfA = rsfA0 if slot == 0 else rsfA1
        rsfB = rsfB0 if slot == 0 else rsfB1
        pltpu.make_async_copy(rsfA, acc.at[rbA.at[slot]], semsc).wait()
        pltpu.make_async_copy(rsfB, acc.at[rbB.at[slot]], semsc).wait()

    def scale_chunk(slot):
        rsb = rsb0 if slot == 0 else rsb1
        rsfA = rsfA0 if slot == 0 else rsfA1
        rsfB = rsfB0 if slot == 0 else rsfB1

        def scale(g, ecarry):
            vv = vbuf[slot, pl.ds(g * 16, 16)]
            for e16 in range(16):
                e = g * 16 + e16
                sv = vv[e16]
                for gg in range(NW):
                    va = plsc.bitcast(
                        rsb[e, pl.ds(gg * 16, 16)], jnp.bfloat16)
                    a, b = plsc.unpack(
                        va, format=plsc.PackFormat.INTERLEAVED)
                    rsf = rsfA if gg < NW // 2 else rsfB
                    g2 = gg if gg < NW // 2 else gg - NW // 2
                    rsf[e, pl.ds(g2 * 32, 16)] = a * sv
                    rsf[e, pl.ds(g2 * 32 + 16, 16)] = b * sv
            return ecarry

        lax.fori_loop(0, K // 16, scale, 0)

    def zinit(i, carry):
        for j in range(128 // 16):
            zbuf[i, pl.ds(j * 16, 16)] = zv
        return carry

    lax.fori_loop(0, RBLK, zinit, 0)

    def zero_blk(b, carry):
        pltpu.sync_copy(zbuf, acc.at[pl.ds(r0base + b * RBLK, RBLK)])
        return carry

    lax.fori_loop(0, NRB, zero_blk, 0)
    plsc.subcore_barrier()

    @pl.when(nj > 0)
    def _():
        start_idx(0, 0)
        wait_idx(0)
        start_gather(0)

    def pipe(i, carry):
        for b in range(2):
            t = 2 * i + b
            wait_gather(b)

            @pl.when(t >= 1)
            def _():
                wait_scatter(1 - b)

            @pl.when(t + 1 < nj)
            def _():
                start_idx(t + 1, 1 - b)
                wait_idx(1 - b)
                start_gather(1 - b)

            scale_chunk(b)
            start_scatter(b)
        return carry

    lax.fori_loop(0, lax.shift_right_logical(nj, 1), pipe, 0)

    @pl.when(nj > 0)
    def _():
        wait_scatter(1)

    plsc.subcore_barrier()

    def readback(b, carry):
        r0 = r0base + b * RBLK
        f0 = f0base + b * RBLK
        t0 = t0base + b * (RBLK // 2)
        pltpu.sync_copy(acc.at[pl.ds(r0, RBLK)], abuf)
        pltpu.sync_copy(fsrc_hbm.at[pl.ds(f0, RBLK)], fbuf)

        def addrow(i, icarry):
            for j in range(128 // 16):
                fbuf[i, pl.ds(j * 16, 16)] = (
                    fbuf[i, pl.ds(j * 16, 16)]
                    + abuf[i, pl.ds(j * 16, 16)])
            return icarry

        lax.fori_loop(0, RBLK, addrow, 0)

        if not last:
            def packrow(i, icarry):
                for gg in range(NW):
                    src_row = 2 * i + (0 if gg < NW // 2 else 1)
                    g2 = gg if gg < NW // 2 else gg - NW // 2
                    a = abuf[src_row, pl.ds(g2 * 32, 16)]
                    bb = abuf[src_row, pl.ds(g2 * 32 + 16, 16)]
                    tbuf[i, pl.ds(gg * 16, 16)] = plsc.bitcast(
                        plsc.pack(
                            a, bb, format=plsc.PackFormat.INTERLEAVED),
                        jnp.int32)
                return icarry

            lax.fori_loop(0, RBLK // 2, packrow, 0)

        pltpu.sync_copy(fbuf, f_hbm.at[pl.ds(f0, RBLK)])
        if not last:
            pltpu.sync_copy(tbuf, tnext_hbm.at[pl.ds(t0, RBLK // 2)])
        return carry

    lax.fori_loop(0, NRB, readback, 0)


_MESH = plsc.VectorSubcoreMesh(core_axis_name="c", subcore_axis_name="s")
_CP = pltpu.CompilerParams(needs_layout_passes=False)


def _make_part():
    return functools.partial(
        pl.kernel,
        mesh=_MESH,
        compiler_params=_CP,
        out_type=(
            jax.ShapeDtypeStruct((NREG,), jnp.int32),
            jax.ShapeDtypeStruct((NREG,), jnp.int32),
            jax.ShapeDtypeStruct((NREG,), jnp.float32),
            jax.ShapeDtypeStruct((2 * NSUB * 16,), jnp.int32),
        ),
        scratch_types=[
            pltpu.VMEM((PB,), jnp.int32),
            pltpu.VMEM((PB,), jnp.int32),
            pltpu.VMEM((PB,), jnp.float32),
            pltpu.VMEM((REG,), jnp.int32),
            pltpu.VMEM((REG,), jnp.int32),
            pltpu.VMEM((REG,), jnp.float32),
            pltpu.VMEM((16,), jnp.int32),
        ],
    )(_part_body)


def _make_layer(last):
    return functools.partial(
        pl.kernel,
        mesh=_MESH,
        compiler_params=_CP,
        out_type=(
            jax.ShapeDtypeStruct((FROWS, 128), jnp.float32),
            jax.ShapeDtypeStruct((NPAD, KI), jnp.int32),
        ),
        scratch_types=[
            pltpu.VMEM_SHARED((2 * NHALF, 128), jnp.float32),
            pltpu.VMEM((16,), jnp.int32),
            pltpu.VMEM((2, K), jnp.int32),
            pltpu.VMEM((2, K), jnp.float32),
            pltpu.VMEM((2, K), jnp.int32),
            pltpu.VMEM((2, K), jnp.int32),
            pltpu.VMEM((2, K), jnp.int32),
            pltpu.VMEM((K, KI), jnp.int32),
            pltpu.VMEM((K, KI), jnp.int32),
            pltpu.VMEM((K, 128), jnp.float32),
            pltpu.VMEM((K, 128), jnp.float32),
            pltpu.VMEM((K, 128), jnp.float32),
            pltpu.VMEM((K, 128), jnp.float32),
            pltpu.VMEM((RBLK, 128), jnp.float32),
            pltpu.VMEM((RBLK, 128), jnp.float32),
            pltpu.VMEM((RBLK // 2, KI), jnp.int32),
            pltpu.VMEM((RBLK, 128), jnp.float32),
            pltpu.SemaphoreType.DMA,
            pltpu.SemaphoreType.DMA,
            pltpu.SemaphoreType.DMA,
        ],
    )(functools.partial(_layer_body, last))


@jax.jit
def _run(rowp, colp, valp, tin, finit):
    part = _make_part()
    mid = _make_layer(False)
    end = _make_layer(True)
    cpk, rpk, vpk, cnt = part(rowp, colp, valp)
    f1, t1 = mid(cnt, cpk, vpk, rpk, tin, finit)
    f2, t2 = mid(cnt, cpk, vpk, rpk, t1, f1)
    f3, _ = end(cnt, cpk, vpk, rpk, t2, f2)
    return f3


def kernel(edge_index, edge_vals, user_embeds, item_embeds):
    embeds = jnp.concatenate([user_embeds, item_embeds], axis=0)
    tstd = jnp.zeros((NPAD, LATDIM), jnp.float32).at[:N_NODES].set(embeds)
    finit = tstd.reshape(FROWS, 128)
    tin_bf = (tstd.reshape(NPAD, NW, 2, 16).transpose(0, 1, 3, 2)
              .reshape(NPAD, LATDIM).astype(jnp.bfloat16))
    tin = jax.lax.bitcast_convert_type(
        tin_bf.reshape(NPAD, KI, 2), jnp.int32)
    pad = EPAD - N_EDGES
    spread = jnp.arange(pad, dtype=jnp.int32) % N_NODES
    rowp = jnp.concatenate([edge_index[0], spread])
    colp = jnp.concatenate([edge_index[1], spread])
    valp = jnp.pad(edge_vals, (0, pad))
    f = _run(rowp, colp, valp, tin, finit)
    final = f[:2 * N_NODES].reshape(N_NODES, LATDIM)
    return final[:USER_NUM], final[USER_NUM:]


# speedup vs baseline: 3.1892x; 1.2475x over previous
import functools

import jax
import jax.numpy as jnp
from jax import lax
from jax.experimental import pallas as pl
from jax.experimental.pallas import tpu as pltpu
from jax.experimental.pallas import tpu_sc as plsc

USER_NUM = 2000
ITEM_NUM = 8000
LATDIM = 256
GCN_LAYER = 3
N_EDGES = 160000
N_NODES = USER_NUM + ITEM_NUM

NSUB = 16
NW = LATDIM // 32
K = 32
KI = LATDIM // 2
NHALF = 5120
NSPLIT = 5000
NPAD = 2 * NHALF
FROWS = 2 * NPAD
RPT = 2 * NHALF // NSUB
RBLK = 32
NRB = RPT // RBLK
EPT = 10000
REG = EPT + 2 * K
NREG = 2 * NSUB * REG
PB = 2000
NPB = EPT // PB


def _part_body(rowp_hbm, colp_hbm, valp_hbm,
               cpk_hbm, rpk_hbm, vpk_hbm, cnt_hbm,
               rin, cin, vin, ccomp, rcomp, vcomp, cw):
    c = lax.axis_index("c")
    s = lax.axis_index("s")
    wid = c * NSUB + s
    lo = c * NSPLIT
    zf = jnp.zeros((16,), jnp.float32)

    def block(bi, off):
        eb = pl.multiple_of(s * EPT + bi * PB, 8)
        pltpu.sync_copy(rowp_hbm.at[pl.ds(eb, PB)], rin)
        pltpu.sync_copy(colp_hbm.at[pl.ds(eb, PB)], cin)
        pltpu.sync_copy(valp_hbm.at[pl.ds(eb, PB)], vin)

        def group(g, off2):
            rv = rin[pl.ds(g * 16, 16)] - lo
            m = (rv >= 0) & (rv < NSPLIT)
            plsc.store_compressed(rcomp.at[pl.ds(off2, 16)], rv, mask=m)
            cvv = cin[pl.ds(g * 16, 16)]
            cvv = cvv + jnp.where(
                cvv >= NSPLIT, jnp.int32(NHALF - NSPLIT), jnp.int32(0))
            plsc.store_compressed(
                ccomp.at[pl.ds(off2, 16)], cvv, mask=m)
            plsc.store_compressed(
                vcomp.at[pl.ds(off2, 16)], vin[pl.ds(g * 16, 16)], mask=m)
            npop = plsc.all_reduce_population_count(m)
            return off2 + npop[0]

        return lax.fori_loop(0, PB // 16, group, off)

    off = lax.fori_loop(0, NPB, block, jnp.int32(0))

    lanes = lax.iota(jnp.int32, 16)
    for gp in range(2 * K // 16):
        spread = lanes + (s * (2 * K) + gp * 16)
        ccomp[pl.ds(off + gp * 16, 16)] = spread
        rcomp[pl.ds(off + gp * 16, 16)] = spread
        vcomp[pl.ds(off + gp * 16, 16)] = zf
    ncp = lax.shift_left(
        lax.shift_right_logical(off + 2 * K - 1, 6), 6)
    cw[pl.ds(0, 16)] = jnp.full((16,), 1, jnp.int32) * ncp

    rb = wid * REG
    pltpu.sync_copy(ccomp, cpk_hbm.at[pl.ds(rb, REG)])
    pltpu.sync_copy(rcomp, rpk_hbm.at[pl.ds(rb, REG)])
    pltpu.sync_copy(vcomp, vpk_hbm.at[pl.ds(rb, REG)])
    pltpu.sync_copy(cw, cnt_hbm.at[pl.ds(wid * 16, 16)])


def _layer_body(last, cnt_hbm, cpk_hbm, vpk_hbm, rpk_hbm, tsrc_hbm, fsrc_hbm,
                f_hbm, tnext_hbm,
                acc, cntbuf, cbuf, vbuf, rbuf, rbA, rbB,
                rsb0, rsb1, rsfA0, rsfA1, rsfB0, rsfB1,
                abuf, fbuf, tbuf, zbuf,
                semi, semg, semsc):
    c = lax.axis_index("c")
    s = lax.axis_index("s")
    wid = c * NSUB + s
    r0base = s * RPT
    f0base = c * 2 * NHALF + s * RPT
    t0base = c * NHALF + s * (RPT // 2)

    zv = jnp.zeros((16,), jnp.float32)

    pltpu.sync_copy(cnt_hbm.at[pl.ds(wid * 16, 16)], cntbuf)
    cv = cntbuf[pl.ds(0, 16)]
    nc = cv[0]
    nj = lax.shift_right_logical(nc, 5)
    ebase = wid * REG

    def start_idx(t, slot):
        eb = pl.multiple_of(ebase + t * K, 8)
        pltpu.async_copy(cpk_hbm.at[pl.ds(eb, K)], cbuf.at[slot], semi)
        pltpu.async_copy(vpk_hbm.at[pl.ds(eb, K)], vbuf.at[slot], semi)
        pltpu.async_copy(rpk_hbm.at[pl.ds(eb, K)], rbuf.at[slot], semi)

    def wait_idx(slot):
        pltpu.make_async_copy(
            cpk_hbm.at[pl.ds(0, K)], cbuf.at[slot], semi).wait()
        pltpu.make_async_copy(
            vpk_hbm.at[pl.ds(0, K)], vbuf.at[slot], semi).wait()
        pltpu.make_async_copy(
            rpk_hbm.at[pl.ds(0, K)], rbuf.at[slot], semi).wait()
        for g in range(K // 16):
            rv = rbuf[slot, pl.ds(g * 16, 16)]
            rbA[slot, pl.ds(g * 16, 16)] = rv * 2
            rbB[slot, pl.ds(g * 16, 16)] = rv * 2 + 1

    def start_gather(slot):
        rsb = rsb0 if slot == 0 else rsb1
        pltpu.async_copy(tsrc_hbm.at[cbuf.at[slot]], rsb, semg)

    def wait_gather(slot):
        rsb = rsb0 if slot == 0 else rsb1
        pltpu.make_async_copy(tsrc_hbm.at[cbuf.at[slot]], rsb, semg).wait()

    def start_scatter(slot):
        rsfA = rsfA0 if slot == 0 else rsfA1
        rsfB = rsfB0 if slot == 0 else rsfB1
        pltpu.async_copy(rsfA, acc.at[rbA.at[slot]], semsc, add=True)
        pltpu.async_copy(rsfB, acc.at[rbB.at[slot]], semsc, add=True)

    def wait_scatter(slot):
        rsfA = rsfA0 if slot == 0 else rsfA1
        rsfB = rsfB0 if slot == 0 else rsfB1
        pltpu.make_async_copy(rsfA, acc.at[rbA.at[slot]], semsc).wait()
        pltpu.make_async_copy(rsfB, acc.at[rbB.at[slot]], semsc).wait()

    def scale_chunk(slot):
        rsb = rsb0 if slot == 0 else rsb1
        rsfA = rsfA0 if slot == 0 else rsfA1
        rsfB = rsfB0 if slot == 0 else rsfB1

        def scale(g, ecarry):
            vv = vbuf[slot, pl.ds(g * 16, 16)]
            for e16 in range(16):
                e = g * 16 + e16
                sv = vv[e16]
                for gg in range(NW):
                    va = plsc.bitcast(
                        rsb[e, pl.ds(gg * 16, 16)], jnp.bfloat16)
                    a, b = plsc.unpack(
                        va, format=plsc.PackFormat.INTERLEAVED)
                    rsf = rsfA if gg < NW // 2 else rsfB
                    g2 = gg if gg < NW // 2 else gg - NW // 2
                    rsf[e, pl.ds(g2 * 32, 16)] = a * sv
                    rsf[e, pl.ds(g2 * 32 + 16, 16)] = b * sv
            return ecarry

        lax.fori_loop(0, K // 16, scale, 0)

    def zinit(i, carry):
        for j in range(128 // 16):
            zbuf[i, pl.ds(j * 16, 16)] = zv
        return carry

    lax.fori_loop(0, RBLK, zinit, 0)

    def zero_blk(b, carry):
        pltpu.sync_copy(zbuf, acc.at[pl.ds(r0base + b * RBLK, RBLK)])
        return carry

    lax.fori_loop(0, NRB, zero_blk, 0)
    plsc.subcore_barrier()

    @pl.when(nj > 0)
    def _():
        start_idx(0, 0)
        wait_idx(0)
        start_gather(0)

    def pipe(i, carry):
        for b in range(2):
            t = 2 * i + b
            wait_gather(b)

            @pl.when(t >= 1)
            def _():
                wait_scatter(1 - b)

            @pl.when(t + 1 < nj)
            def _():
                start_idx(t + 1, 1 - b)
                wait_idx(1 - b)
                start_gather(1 - b)

            scale_chunk(b)
            start_scatter(b)
        return carry

    lax.fori_loop(0, lax.shift_right_logical(nj, 1), pipe, 0)

    @pl.when(nj > 0)
    def _():
        wait_scatter(1)

    plsc.subcore_barrier()

    def readback(b, carry):
        r0 = r0base + b * RBLK
        f0 = f0base + b * RBLK
        t0 = t0base + b * (RBLK // 2)
        pltpu.sync_copy(acc.at[pl.ds(r0, RBLK)], abuf)
        pltpu.sync_copy(fsrc_hbm.at[pl.ds(f0, RBLK)], fbuf)

        def addrow(i, icarry):
            for j in range(128 // 16):
                fbuf[i, pl.ds(j * 16, 16)] = (
                    fbuf[i, pl.ds(j * 16, 16)]
                    + abuf[i, pl.ds(j * 16, 16)])
            return icarry

        lax.fori_loop(0, RBLK, addrow, 0)

        if not last:
            def packrow(i, icarry):
                for gg in range(NW):
                    src_row = 2 * i + (0 if gg < NW // 2 else 1)
                    g2 = gg if gg < NW // 2 else gg - NW // 2
                    a = abuf[src_row, pl.ds(g2 * 32, 16)]
                    bb = abuf[src_row, pl.ds(g2 * 32 + 16, 16)]
                    tbuf[i, pl.ds(gg * 16, 16)] = plsc.bitcast(
                        plsc.pack(
                            a, bb, format=plsc.PackFormat.INTERLEAVED),
                        jnp.int32)
                return icarry

            lax.fori_loop(0, RBLK // 2, packrow, 0)

        pltpu.sync_copy(fbuf, f_hbm.at[pl.ds(f0, RBLK)])
        if not last:
            pltpu.sync_copy(tbuf, tnext_hbm.at[pl.ds(t0, RBLK // 2)])
        return carry

    lax.fori_loop(0, NRB, readback, 0)


_MESH = plsc.VectorSubcoreMesh(core_axis_name="c", subcore_axis_name="s")
_CP = pltpu.CompilerParams(needs_layout_passes=False)


def _make_part():
    return functools.partial(
        pl.kernel,
        mesh=_MESH,
        compiler_params=_CP,
        out_type=(
            jax.ShapeDtypeStruct((NREG,), jnp.int32),
            jax.ShapeDtypeStruct((NREG,), jnp.int32),
            jax.ShapeDtypeStruct((NREG,), jnp.float32),
            jax.ShapeDtypeStruct((2 * NSUB * 16,), jnp.int32),
        ),
        scratch_types=[
            pltpu.VMEM((PB,), jnp.int32),
            pltpu.VMEM((PB,), jnp.int32),
            pltpu.VMEM((PB,), jnp.float32),
            pltpu.VMEM((REG,), jnp.int32),
            pltpu.VMEM((REG,), jnp.int32),
            pltpu.VMEM((REG,), jnp.float32),
            pltpu.VMEM((16,), jnp.int32),
        ],
    )(_part_body)


def _make_layer(last):
    return functools.partial(
        pl.kernel,
        mesh=_MESH,
        compiler_params=_CP,
        out_type=(
            jax.ShapeDtypeStruct((FROWS, 128), jnp.float32),
            jax.ShapeDtypeStruct((NPAD, KI), jnp.int32),
        ),
        scratch_types=[
            pltpu.VMEM_SHARED((2 * NHALF, 128), jnp.float32),
            pltpu.VMEM((16,), jnp.int32),
            pltpu.VMEM((2, K), jnp.int32),
            pltpu.VMEM((2, K), jnp.float32),
            pltpu.VMEM((2, K), jnp.int32),
            pltpu.VMEM((2, K), jnp.int32),
            pltpu.VMEM((2, K), jnp.int32),
            pltpu.VMEM((K, KI), jnp.int32),
            pltpu.VMEM((K, KI), jnp.int32),
            pltpu.VMEM((K, 128), jnp.float32),
            pltpu.VMEM((K, 128), jnp.float32),
            pltpu.VMEM((K, 128), jnp.float32),
            pltpu.VMEM((K, 128), jnp.float32),
            pltpu.VMEM((RBLK, 128), jnp.float32),
            pltpu.VMEM((RBLK, 128), jnp.float32),
            pltpu.VMEM((RBLK // 2, KI), jnp.int32),
            pltpu.VMEM((RBLK, 128), jnp.float32),
            pltpu.SemaphoreType.DMA,
            pltpu.SemaphoreType.DMA,
            pltpu.SemaphoreType.DMA,
        ],
    )(functools.partial(_layer_body, last))


@jax.jit
def _run(rowp, colp, valp, tin, finit):
    part = _make_part()
    mid = _make_layer(False)
    end = _make_layer(True)
    cpk, rpk, vpk, cnt = part(rowp, colp, valp)
    f1, t1 = mid(cnt, cpk, vpk, rpk, tin, finit)
    f2, t2 = mid(cnt, cpk, vpk, rpk, t1, f1)
    f3, _ = end(cnt, cpk, vpk, rpk, t2, f2)
    return f3


def kernel(edge_index, edge_vals, user_embeds, item_embeds):
    embeds = jnp.concatenate([user_embeds, item_embeds], axis=0)
    tstd = (jnp.zeros((NPAD, LATDIM), jnp.float32)
            .at[:NSPLIT].set(embeds[:NSPLIT])
            .at[NHALF:NHALF + N_NODES - NSPLIT].set(embeds[NSPLIT:]))
    finit = tstd.reshape(FROWS, 128)
    tin_bf = (tstd.reshape(NPAD, NW, 2, 16).transpose(0, 1, 3, 2)
              .reshape(NPAD, LATDIM).astype(jnp.bfloat16))
    tin = jax.lax.bitcast_convert_type(
        tin_bf.reshape(NPAD, KI, 2), jnp.int32)
    f = _run(edge_index[0], edge_index[1], edge_vals, tin, finit)
    final = jnp.concatenate(
        [f[:2 * NSPLIT].reshape(NSPLIT, LATDIM),
         f[2 * NHALF:2 * NHALF + 2 * (N_NODES - NSPLIT)].reshape(
             N_NODES - NSPLIT, LATDIM)], axis=0)
    return final[:USER_NUM], final[USER_NUM:]
